# Initial kernel scaffold; baseline (speedup 1.0000x reference)
import jax
import jax.numpy as jnp
from jax.experimental import pallas as pl


def kernel(x, edge_index, W1, b1, W2, b2, W_reg, b_reg):
    raise NotImplementedError("write your pallas kernel here")


# speedup vs baseline: 4.4164x; 4.4164x over previous
import functools

import jax
import jax.numpy as jnp
from jax import lax
from jax.experimental import pallas as pl
from jax.experimental.pallas import tpu as pltpu
from jax.experimental.pallas import tpu_sc as plsc

N = 10000
E = 320000
D_IN = 128
H1 = 256
H2 = 128

NC = 2
NS = 16
NW = NC * NS
EPW = E // NW
CH = 80
NCHUNK = EPW // CH
RPS = 624
TAIL0 = (NS - 1) * RPS
TAILN = N - TAIL0


def _sc_mesh():
    return plsc.VectorSubcoreMesh(core_axis_name="c", subcore_axis_name="s",
                                  num_cores=NC, num_subcores=NS)


NCH2 = E // NS // CH


def _sc_degrees(ids4, zrows, ones_rows):
    @functools.partial(
        pl.kernel,
        out_type=jax.ShapeDtypeStruct((2, N, D_IN), jnp.float32),
        mesh=_sc_mesh(),
        scratch_types=[
            pltpu.VMEM((NCH2, CH), jnp.int32),
            pltpu.VMEM((CH, D_IN), jnp.float32),
            pltpu.VMEM_SHARED((N, D_IN), jnp.float32),
        ],
    )
    def deg_kernel(ids_hbm, z_hbm, ones_hbm, out_hbm, ids_v, ones_v, acc_sh):
        c = lax.axis_index("c")
        s = lax.axis_index("s")
        pltpu.sync_copy(ids_hbm.at[c, s], ids_v)
        pltpu.sync_copy(ones_hbm, ones_v)

        @pl.when(s < NS - 1)
        def _():
            pltpu.sync_copy(z_hbm.at[pl.ds(s * RPS, RPS)],
                            acc_sh.at[pl.ds(s * RPS, RPS)])

        @pl.when(s == NS - 1)
        def _():
            pltpu.sync_copy(z_hbm.at[pl.ds(TAIL0, TAILN)],
                            acc_sh.at[pl.ds(TAIL0, TAILN)])

        plsc.subcore_barrier()

        def body(j, carry):
            pltpu.sync_copy(ones_v, acc_sh.at[ids_v.at[j]], add=True)
            return carry

        lax.fori_loop(0, NCH2, body, 0)

        plsc.subcore_barrier()

        @pl.when(s < NS - 1)
        def _():
            pltpu.sync_copy(acc_sh.at[pl.ds(s * RPS, RPS)],
                            out_hbm.at[c, pl.ds(s * RPS, RPS)])

        @pl.when(s == NS - 1)
        def _():
            pltpu.sync_copy(acc_sh.at[pl.ds(TAIL0, TAILN)],
                            out_hbm.at[c, pl.ds(TAIL0, TAILN)])

    return deg_kernel(ids4, zrows, ones_rows)


def _sc_aggregate(table, src_c, dst_c, zrows):
    @functools.partial(
        pl.kernel,
        out_type=jax.ShapeDtypeStruct((NC, N, D_IN), jnp.float32),
        mesh=_sc_mesh(),
        scratch_types=[
            pltpu.VMEM((NCHUNK, CH), jnp.int32),
            pltpu.VMEM((NCHUNK, CH), jnp.int32),
            pltpu.VMEM((CH, D_IN), jnp.float32),
            pltpu.VMEM_SHARED((N, D_IN), jnp.float32),
            pltpu.SemaphoreType.DMA,
        ],
    )
    def agg_kernel(tab_hbm, src_hbm, dst_hbm, z_hbm, out_hbm,
                   src_v, dst_v, rows_v, acc_sh, sem):
        c = lax.axis_index("c")
        s = lax.axis_index("s")
        wid = s * NC + c
        pltpu.sync_copy(src_hbm.at[wid], src_v)
        pltpu.sync_copy(dst_hbm.at[wid], dst_v)

        @pl.when(s < NS - 1)
        def _():
            pltpu.sync_copy(z_hbm.at[pl.ds(s * RPS, RPS)],
                            acc_sh.at[pl.ds(s * RPS, RPS)])

        @pl.when(s == NS - 1)
        def _():
            pltpu.sync_copy(z_hbm.at[pl.ds(TAIL0, TAILN)],
                            acc_sh.at[pl.ds(TAIL0, TAILN)])

        plsc.subcore_barrier()

        def body(j, carry):
            pltpu.async_copy(tab_hbm.at[src_v.at[j]], rows_v, sem).wait()
            pltpu.sync_copy(rows_v, acc_sh.at[dst_v.at[j]], add=True)
            return carry

        lax.fori_loop(0, NCHUNK, body, 0)

        plsc.subcore_barrier()

        @pl.when(s < NS - 1)
        def _():
            pltpu.sync_copy(acc_sh.at[pl.ds(s * RPS, RPS)],
                            out_hbm.at[c, pl.ds(s * RPS, RPS)])

        @pl.when(s == NS - 1)
        def _():
            pltpu.sync_copy(acc_sh.at[pl.ds(TAIL0, TAILN)],
                            out_hbm.at[c, pl.ds(TAIL0, TAILN)])

    return agg_kernel(table, src_c, dst_c, zrows)


def _tc_prep(deg_parts, x):
    def body(dp_ref, x_ref, h0_ref, norms_ref):
        deg = dp_ref[...][:, :, 0]
        norm = lax.rsqrt(jnp.maximum(deg, 1.0))
        norms_ref[...] = norm.T
        h0_ref[...] = x_ref[...] * norm[0][:, None]

    return pl.pallas_call(
        body,
        out_shape=[
            jax.ShapeDtypeStruct((N, D_IN), jnp.float32),
            jax.ShapeDtypeStruct((N, 2), jnp.float32),
        ],
    )(deg_parts, x)


def _tc_mlp(parts, norms, W1, b1, W2):
    BR = 1000

    def body(p_ref, n_ref, w1_ref, b1_ref, w2_ref, y_ref):
        a = p_ref[0] + p_ref[1]
        ns = n_ref[:, 0:1]
        nd = n_ref[:, 1:2]
        h1 = jnp.dot(a * nd, w1_ref[...], preferred_element_type=jnp.float32)
        h1 = jnp.maximum(h1 + b1_ref[...], 0.0)
        y_ref[...] = jnp.dot(h1 * ns, w2_ref[...],
                             preferred_element_type=jnp.float32)

    return pl.pallas_call(
        body,
        grid=(N // BR,),
        in_specs=[
            pl.BlockSpec((2, BR, D_IN), lambda i: (0, i, 0)),
            pl.BlockSpec((BR, 2), lambda i: (i, 0)),
            pl.BlockSpec((D_IN, H1), lambda i: (0, 0)),
            pl.BlockSpec((1, H1), lambda i: (0, 0)),
            pl.BlockSpec((H1, H2), lambda i: (0, 0)),
        ],
        out_specs=pl.BlockSpec((BR, H2), lambda i: (i, 0)),
        out_shape=jax.ShapeDtypeStruct((N, H2), jnp.float32),
    )(parts, norms, W1, b1.reshape(1, H1), W2)


def _tc_head(parts, norms, b2, W_reg, b_reg):
    def body(p_ref, n_ref, b2_ref, wr_ref, br_ref, h2_ref, pred_ref):
        a = p_ref[0] + p_ref[1]
        nd = n_ref[:, 1:2]
        h2 = jnp.maximum(a * nd + b2_ref[...], 0.0)
        h2_ref[...] = h2
        hg = jnp.mean(h2, axis=0, keepdims=True)
        pred_ref[...] = jnp.dot(hg, wr_ref[...],
                                preferred_element_type=jnp.float32) + br_ref[...]

    return pl.pallas_call(
        body,
        out_shape=[
            jax.ShapeDtypeStruct((N, H2), jnp.float32),
            jax.ShapeDtypeStruct((1, 1), jnp.float32),
        ],
    )(parts, norms, b2.reshape(1, H2), W_reg, b_reg.reshape(1, 1))


def _tc_decoder(h2):
    B = 512
    G = pl.cdiv(N, B)

    def body(a_ref, b_ref, o_ref):
        prod = lax.dot_general(a_ref[...], b_ref[...],
                               (((1,), (1,)), ((), ())),
                               preferred_element_type=jnp.float32)
        o_ref[...] = jax.nn.sigmoid(prod)

    return pl.pallas_call(
        body,
        grid=(G, G),
        in_specs=[
            pl.BlockSpec((B, H2), lambda i, j: (i, 0)),
            pl.BlockSpec((B, H2), lambda i, j: (j, 0)),
        ],
        out_specs=pl.BlockSpec((B, B), lambda i, j: (i, j)),
        out_shape=jax.ShapeDtypeStruct((N, N), jnp.float32),
    )(h2, h2)


def kernel(x, edge_index, W1, b1, W2, b2, W_reg, b_reg):
    src = edge_index[0]
    dst = edge_index[1]
    src_c = src.reshape(NW, NCHUNK, CH)
    dst_c = dst.reshape(NW, NCHUNK, CH)
    ids4 = edge_index.reshape(2, NS, NCH2, CH)
    zrows = jnp.zeros((N, D_IN), jnp.float32)
    ones_rows = jnp.ones((CH, D_IN), jnp.float32)

    deg_parts = _sc_degrees(ids4, zrows, ones_rows)
    h0, norms = _tc_prep(deg_parts, x)
    agg1 = _sc_aggregate(h0, src_c, dst_c, zrows)
    y = _tc_mlp(agg1, norms, W1, b1, W2)
    agg2 = _sc_aggregate(y, src_c, dst_c, zrows)
    h2, pred = _tc_head(agg2, norms, b2, W_reg, b_reg)
    reconstructed = _tc_decoder(h2)
    return (reconstructed, pred)


# speedup vs baseline: 4.5158x; 1.0225x over previous
import functools

import jax
import jax.numpy as jnp
from jax import lax
from jax.experimental import pallas as pl
from jax.experimental.pallas import tpu as pltpu
from jax.experimental.pallas import tpu_sc as plsc

N = 10000
E = 320000
D_IN = 128
H1 = 256
H2 = 128

NC = 2
NS = 16
NW = NC * NS
EPW = E // NW
CH = 80
NCHUNK = EPW // CH
CHA = 80
NCHA = EPW // CHA
RPS = 624
TAIL0 = (NS - 1) * RPS
TAILN = N - TAIL0


def _sc_mesh():
    return plsc.VectorSubcoreMesh(core_axis_name="c", subcore_axis_name="s",
                                  num_cores=NC, num_subcores=NS)


NCH2 = E // NS // CH


def _sc_degrees(ids4, zrows, ones_rows):
    @functools.partial(
        pl.kernel,
        out_type=jax.ShapeDtypeStruct((2, N, D_IN), jnp.float32),
        mesh=_sc_mesh(),
        scratch_types=[
            pltpu.VMEM((NCH2, CH), jnp.int32),
            pltpu.VMEM((CH, D_IN), jnp.float32),
            pltpu.VMEM_SHARED((N, D_IN), jnp.float32),
            pltpu.SemaphoreType.DMA,
        ],
    )
    def deg_kernel(ids_hbm, z_hbm, ones_hbm, out_hbm, ids_v, ones_v, acc_sh,
                   sem):
        c = lax.axis_index("c")
        s = lax.axis_index("s")
        pltpu.sync_copy(ids_hbm.at[c, s], ids_v)
        pltpu.sync_copy(ones_hbm, ones_v)

        @pl.when(s < NS - 1)
        def _():
            pltpu.sync_copy(z_hbm.at[pl.ds(s * RPS, RPS)],
                            acc_sh.at[pl.ds(s * RPS, RPS)])

        @pl.when(s == NS - 1)
        def _():
            pltpu.sync_copy(z_hbm.at[pl.ds(TAIL0, TAILN)],
                            acc_sh.at[pl.ds(TAIL0, TAILN)])

        plsc.subcore_barrier()

        def body(j, carry):
            pltpu.sync_copy(ones_v, acc_sh.at[ids_v.at[j]], add=True)
            return carry

        lax.fori_loop(0, NCH2, body, 0)

        plsc.subcore_barrier()

        @pl.when(s < NS - 1)
        def _():
            pltpu.sync_copy(acc_sh.at[pl.ds(s * RPS, RPS)],
                            out_hbm.at[c, pl.ds(s * RPS, RPS)])

        @pl.when(s == NS - 1)
        def _():
            pltpu.sync_copy(acc_sh.at[pl.ds(TAIL0, TAILN)],
                            out_hbm.at[c, pl.ds(TAIL0, TAILN)])

    return deg_kernel(ids4, zrows, ones_rows)


def _sc_aggregate(table, src_c, dst_c, zrows):
    @functools.partial(
        pl.kernel,
        out_type=jax.ShapeDtypeStruct((NC, N, D_IN), jnp.float32),
        mesh=_sc_mesh(),
        scratch_types=[
            pltpu.VMEM((NCHA, CHA), jnp.int32),
            pltpu.VMEM((NCHA, CHA), jnp.int32),
            pltpu.VMEM((CHA, D_IN), jnp.float32),
            pltpu.VMEM_SHARED((N, D_IN), jnp.float32),
            pltpu.SemaphoreType.DMA,
        ],
    )
    def agg_kernel(tab_hbm, src_hbm, dst_hbm, z_hbm, out_hbm,
                   src_v, dst_v, rows_v, acc_sh, sem):
        c = lax.axis_index("c")
        s = lax.axis_index("s")
        wid = s * NC + c
        pltpu.sync_copy(src_hbm.at[wid], src_v)
        pltpu.sync_copy(dst_hbm.at[wid], dst_v)

        @pl.when(s < NS - 1)
        def _():
            pltpu.sync_copy(z_hbm.at[pl.ds(s * RPS, RPS)],
                            acc_sh.at[pl.ds(s * RPS, RPS)])

        @pl.when(s == NS - 1)
        def _():
            pltpu.sync_copy(z_hbm.at[pl.ds(TAIL0, TAILN)],
                            acc_sh.at[pl.ds(TAIL0, TAILN)])

        plsc.subcore_barrier()

        def body(j, carry):
            pltpu.async_copy(tab_hbm.at[src_v.at[j]], rows_v, sem).wait()
            pltpu.sync_copy(rows_v, acc_sh.at[dst_v.at[j]], add=True)
            return carry

        lax.fori_loop(0, NCHA, body, 0)

        plsc.subcore_barrier()

        @pl.when(s < NS - 1)
        def _():
            pltpu.sync_copy(acc_sh.at[pl.ds(s * RPS, RPS)],
                            out_hbm.at[c, pl.ds(s * RPS, RPS)])

        @pl.when(s == NS - 1)
        def _():
            pltpu.sync_copy(acc_sh.at[pl.ds(TAIL0, TAILN)],
                            out_hbm.at[c, pl.ds(TAIL0, TAILN)])

    return agg_kernel(table, src_c, dst_c, zrows)


def _tc_prep(deg_parts, x):
    def body(dp_ref, x_ref, h0_ref, norms_ref):
        deg = dp_ref[...][:, :, 0]
        norm = lax.rsqrt(jnp.maximum(deg, 1.0))
        norms_ref[...] = norm.T
        h0_ref[...] = x_ref[...] * norm[0][:, None]

    return pl.pallas_call(
        body,
        out_shape=[
            jax.ShapeDtypeStruct((N, D_IN), jnp.float32),
            jax.ShapeDtypeStruct((N, 2), jnp.float32),
        ],
    )(deg_parts, x)


def _tc_mlp(parts, norms, W1, b1, W2):
    BR = 1000

    def body(p_ref, n_ref, w1_ref, b1_ref, w2_ref, y_ref):
        a = p_ref[0] + p_ref[1]
        ns = n_ref[:, 0:1]
        nd = n_ref[:, 1:2]
        h1 = jnp.dot(a * nd, w1_ref[...], preferred_element_type=jnp.float32)
        h1 = jnp.maximum(h1 + b1_ref[...], 0.0)
        y_ref[...] = jnp.dot(h1 * ns, w2_ref[...],
                             preferred_element_type=jnp.float32)

    return pl.pallas_call(
        body,
        grid=(N // BR,),
        in_specs=[
            pl.BlockSpec((2, BR, D_IN), lambda i: (0, i, 0)),
            pl.BlockSpec((BR, 2), lambda i: (i, 0)),
            pl.BlockSpec((D_IN, H1), lambda i: (0, 0)),
            pl.BlockSpec((1, H1), lambda i: (0, 0)),
            pl.BlockSpec((H1, H2), lambda i: (0, 0)),
        ],
        out_specs=pl.BlockSpec((BR, H2), lambda i: (i, 0)),
        out_shape=jax.ShapeDtypeStruct((N, H2), jnp.float32),
    )(parts, norms, W1, b1.reshape(1, H1), W2)


def _tc_head(parts, norms, b2, W_reg, b_reg):
    def body(p_ref, n_ref, b2_ref, wr_ref, br_ref, h2_ref, pred_ref):
        a = p_ref[0] + p_ref[1]
        nd = n_ref[:, 1:2]
        h2 = jnp.maximum(a * nd + b2_ref[...], 0.0)
        h2_ref[...] = h2
        hg = jnp.mean(h2, axis=0, keepdims=True)
        pred_ref[...] = jnp.dot(hg, wr_ref[...],
                                preferred_element_type=jnp.float32) + br_ref[...]

    return pl.pallas_call(
        body,
        out_shape=[
            jax.ShapeDtypeStruct((N, H2), jnp.float32),
            jax.ShapeDtypeStruct((1, 1), jnp.float32),
        ],
    )(parts, norms, b2.reshape(1, H2), W_reg, b_reg.reshape(1, 1))


def _tc_decoder(h2):
    B = 512
    G = pl.cdiv(N, B)

    def body(a_ref, b_ref, o_ref):
        prod = lax.dot_general(a_ref[...], b_ref[...],
                               (((1,), (1,)), ((), ())),
                               preferred_element_type=jnp.float32)
        o_ref[...] = 0.5 * jnp.tanh(0.5 * prod) + 0.5

    return pl.pallas_call(
        body,
        grid=(G, G),
        in_specs=[
            pl.BlockSpec((B, H2), lambda i, j: (i, 0)),
            pl.BlockSpec((B, H2), lambda i, j: (j, 0)),
        ],
        out_specs=pl.BlockSpec((B, B), lambda i, j: (i, j)),
        out_shape=jax.ShapeDtypeStruct((N, N), jnp.float32),
    )(h2, h2)


def kernel(x, edge_index, W1, b1, W2, b2, W_reg, b_reg):
    src = edge_index[0]
    dst = edge_index[1]
    src_c = src.reshape(NW, NCHA, CHA)
    dst_c = dst.reshape(NW, NCHA, CHA)
    ids4 = edge_index.reshape(2, NS, NCH2, CH)
    zrows = jnp.zeros((N, D_IN), jnp.float32)
    ones_rows = jnp.ones((CH, D_IN), jnp.float32)

    deg_parts = _sc_degrees(ids4, zrows, ones_rows)
    h0, norms = _tc_prep(deg_parts, x)
    agg1 = _sc_aggregate(h0, src_c, dst_c, zrows)
    y = _tc_mlp(agg1, norms, W1, b1, W2)
    agg2 = _sc_aggregate(y, src_c, dst_c, zrows)
    h2, pred = _tc_head(agg2, norms, b2, W_reg, b_reg)
    reconstructed = _tc_decoder(h2)
    return (reconstructed, pred)
